# baseline (speedup 1.0000x reference)
import functools

import jax
import jax.numpy as jnp
from jax import lax
from jax.experimental import pallas as pl
from jax.experimental.pallas import tpu as pltpu
from jax.experimental.pallas import tpu_sc as plsc

NUM_SEG = 16
TOTAL_TOK = 32768
D = 1024
L = 16
NC = 2
NS = 16
NW = NC * NS
RPW = TOTAL_TOK // NW
C = 16
NCH = RPW // C
NDC = D // L
ACC = NUM_SEG * D

_mesh = plsc.VectorSubcoreMesh(core_axis_name="c", subcore_axis_name="s")


@functools.partial(
    pl.kernel,
    mesh=_mesh,
    out_type=[
        jax.ShapeDtypeStruct((NW, ACC), jnp.float32),
        jax.ShapeDtypeStruct((NW, ACC), jnp.float32),
    ],
    scratch_types=[
        pltpu.VMEM((2, C, D), jnp.float32),
        pltpu.VMEM((2, C, D), jnp.float32),
        pltpu.VMEM((RPW,), jnp.int32),
        pltpu.VMEM((ACC,), jnp.float32),
        pltpu.VMEM((ACC,), jnp.float32),
        pltpu.SemaphoreType.DMA,
        pltpu.SemaphoreType.DMA,
        pltpu.SemaphoreType.DMA,
        pltpu.SemaphoreType.DMA,
    ],
    compiler_params=pltpu.CompilerParams(needs_layout_passes=False),
)
def _seg_sumsq(inp, tgt, idx, outd, outt, bufa, buft, idxv, accd, acct,
               sa0, sa1, st0, st1):
    wid = lax.axis_index("s") * NC + lax.axis_index("c")
    base = wid * RPW
    sems_a = (sa0, sa1)
    sems_t = (st0, st1)

    pltpu.sync_copy(idx.at[pl.ds(base, RPW)], idxv)

    zero = jnp.zeros((L,), jnp.float32)
    lane = lax.iota(jnp.int32, L)

    def _z(i, carry):
        accd[pl.ds(i * L, L)] = zero
        acct[pl.ds(i * L, L)] = zero
        return carry

    lax.fori_loop(0, ACC // L, _z, 0, unroll=8)

    def _start(ch, b):
        r0 = base + ch * C
        pltpu.make_async_copy(inp.at[pl.ds(r0, C), :], bufa.at[b],
                              sems_a[b]).start()
        pltpu.make_async_copy(tgt.at[pl.ds(r0, C), :], buft.at[b],
                              sems_t[b]).start()

    def _wait(b):
        pltpu.make_async_copy(inp.at[pl.ds(base, C), :], bufa.at[b],
                              sems_a[b]).wait()
        pltpu.make_async_copy(tgt.at[pl.ds(base, C), :], buft.at[b],
                              sems_t[b]).wait()

    _start(0, 0)
    _start(1, 1)

    def _chunk(ch, b):
        _wait(b)
        vi = idxv[pl.ds(ch * C, L)]
        s0 = vi[0]
        s15 = vi[L - 1]

        @pl.when(s0 == s15)
        def _fast():
            off = s0 * D

            def _dc(dc, carry):
                ad = [zero] * 4
                at = [zero] * 4
                for r in range(C):
                    av = bufa[b, r, pl.ds(dc * L, L)]
                    tv = buft[b, r, pl.ds(dc * L, L)]
                    dv = av - tv
                    k = r % 4
                    ad[k] = ad[k] + dv * dv
                    at[k] = at[k] + tv * tv
                accd[pl.ds(off + dc * L, L)] += (ad[0] + ad[1]) + (ad[2] + ad[3])
                acct[pl.ds(off + dc * L, L)] += (at[0] + at[1]) + (at[2] + at[3])
                return carry

            lax.fori_loop(0, NDC, _dc, 0)

        @pl.when(s0 != s15)
        def _slow():
            for r in range(C):
                sr = vi[r]
                off = sr * D

                def _dc2(dc, carry):
                    av = bufa[b, r, pl.ds(dc * L, L)]
                    tv = buft[b, r, pl.ds(dc * L, L)]
                    dv = av - tv
                    accd[pl.ds(off + dc * L, L)] += dv * dv
                    acct[pl.ds(off + dc * L, L)] += tv * tv
                    return carry

                lax.fori_loop(0, NDC, _dc2, 0, unroll=4)

        @pl.when(ch + 2 < NCH)
        def _next():
            _start(ch + 2, b)

    def _outer(g, carry):
        _chunk(g * 2, 0)
        _chunk(g * 2 + 1, 1)
        return carry

    lax.fori_loop(0, NCH // 2, _outer, 0)

    pltpu.sync_copy(accd, outd.at[wid])
    pltpu.sync_copy(acct, outt.at[wid])


def _epi_body(pd_ref, pt_ref, o_ref):
    sd = jnp.sum(pd_ref[...], axis=0)
    st = jnp.sum(pt_ref[...], axis=0)
    dn = jnp.sqrt(sd)
    tn = jnp.maximum(jnp.sqrt(st), 1.0)
    o_ref[0, 0] = jnp.mean(dn / tn)


def _epilogue(pd, pt):
    return pl.pallas_call(
        _epi_body,
        out_specs=pl.BlockSpec(memory_space=pltpu.SMEM),
        out_shape=jax.ShapeDtypeStruct((1, 1), jnp.float32),
    )(pd, pt)


def kernel(input, target, batch_idx):
    pd, pt = _seg_sumsq(input, target, batch_idx.astype(jnp.int32))
    return _epilogue(pd, pt)[0, 0]


# speedup vs baseline: 1.8440x; 1.8440x over previous
import functools

import jax
import jax.numpy as jnp
from jax import lax
from jax.experimental import pallas as pl
from jax.experimental.pallas import tpu as pltpu
from jax.experimental.pallas import tpu_sc as plsc

NUM_SEG = 16
TOTAL_TOK = 32768
D = 1024
L = 16
NC = 2
NS = 16
NW = NC * NS
C = 16
NDC = D // L
ACC = NUM_SEG * D

R_SC = 8192
R_TC = TOTAL_TOK - R_SC
RPW = R_SC // NW
NCH = RPW // C
BT = 2048
NB_TC = R_TC // BT
B0_TC = R_SC // BT

_mesh = plsc.VectorSubcoreMesh(core_axis_name="c", subcore_axis_name="s")


@functools.partial(
    pl.kernel,
    mesh=_mesh,
    out_type=[
        jax.ShapeDtypeStruct((NW, ACC), jnp.float32),
        jax.ShapeDtypeStruct((NW, ACC), jnp.float32),
    ],
    scratch_types=[
        pltpu.VMEM((2, C, D), jnp.float32),
        pltpu.VMEM((2, C, D), jnp.float32),
        pltpu.VMEM((RPW,), jnp.int32),
        pltpu.VMEM((ACC,), jnp.float32),
        pltpu.VMEM((ACC,), jnp.float32),
        pltpu.SemaphoreType.DMA,
        pltpu.SemaphoreType.DMA,
        pltpu.SemaphoreType.DMA,
        pltpu.SemaphoreType.DMA,
    ],
    compiler_params=pltpu.CompilerParams(needs_layout_passes=False),
)
def _seg_sumsq_sc(inp, tgt, idx, outd, outt, bufa, buft, idxv, accd, acct,
                  sa0, sa1, st0, st1):
    wid = lax.axis_index("s") * NC + lax.axis_index("c")
    base = wid * RPW
    sems_a = (sa0, sa1)
    sems_t = (st0, st1)

    pltpu.sync_copy(idx.at[pl.ds(base, RPW)], idxv)

    zero = jnp.zeros((L,), jnp.float32)

    def _z(i, carry):
        accd[pl.ds(i * L, L)] = zero
        acct[pl.ds(i * L, L)] = zero
        return carry

    lax.fori_loop(0, ACC // L, _z, 0, unroll=8)

    def _start(ch, b):
        r0 = base + ch * C
        pltpu.make_async_copy(inp.at[pl.ds(r0, C), :], bufa.at[b],
                              sems_a[b]).start()
        pltpu.make_async_copy(tgt.at[pl.ds(r0, C), :], buft.at[b],
                              sems_t[b]).start()

    def _wait(b):
        pltpu.make_async_copy(inp.at[pl.ds(base, C), :], bufa.at[b],
                              sems_a[b]).wait()
        pltpu.make_async_copy(tgt.at[pl.ds(base, C), :], buft.at[b],
                              sems_t[b]).wait()

    _start(0, 0)
    _start(1, 1)

    def _chunk(ch, b):
        _wait(b)
        vi = idxv[pl.ds(ch * C, L)]
        s0 = vi[0]
        s15 = vi[L - 1]

        @pl.when(s0 == s15)
        def _fast():
            off = s0 * D

            def _dc(dc, carry):
                ad = zero
                at = zero
                for r in range(C):
                    av = bufa[b, r, pl.ds(dc * L, L)]
                    tv = buft[b, r, pl.ds(dc * L, L)]
                    dv = av - tv
                    ad = ad + dv * dv
                    at = at + tv * tv
                accd[pl.ds(off + dc * L, L)] += ad
                acct[pl.ds(off + dc * L, L)] += at
                return carry

            lax.fori_loop(0, NDC, _dc, 0)

        @pl.when(s0 != s15)
        def _slow():
            for r in range(C):
                sr = vi[r]
                off = sr * D

                def _dc2(dc, carry):
                    av = bufa[b, r, pl.ds(dc * L, L)]
                    tv = buft[b, r, pl.ds(dc * L, L)]
                    dv = av - tv
                    accd[pl.ds(off + dc * L, L)] += dv * dv
                    acct[pl.ds(off + dc * L, L)] += tv * tv
                    return carry

                lax.fori_loop(0, NDC, _dc2, 0, unroll=4)

        @pl.when(ch + 2 < NCH)
        def _next():
            _start(ch + 2, b)

    def _outer(g, carry):
        _chunk(g * 2, 0)
        _chunk(g * 2 + 1, 1)
        return carry

    lax.fori_loop(0, NCH // 2, _outer, 0)

    pltpu.sync_copy(accd, outd.at[wid])
    pltpu.sync_copy(acct, outt.at[wid])


def _tc_body(idx_ref, x_ref, t_ref, od_ref, ot_ref, acc_d, acc_t):
    i = pl.program_id(0)

    idx = idx_ref[0, 0, :]
    onehot = (jax.lax.broadcasted_iota(jnp.int32, (NUM_SEG, BT), 0)
              == idx[None, :]).astype(jnp.float32)

    x = x_ref[...]
    t = t_ref[...]
    d = x - t
    pd = jnp.dot(onehot, d * d, preferred_element_type=jnp.float32)
    pt = jnp.dot(onehot, t * t, preferred_element_type=jnp.float32)

    @pl.when(i == 0)
    def _init():
        acc_d[...] = pd
        acc_t[...] = pt

    @pl.when(i > 0)
    def _accum():
        acc_d[...] += pd
        acc_t[...] += pt

    @pl.when(i == NB_TC - 1)
    def _fin():
        od_ref[...] = acc_d[...]
        ot_ref[...] = acc_t[...]


def _seg_sumsq_tc(inp, tgt, idx3):
    return pl.pallas_call(
        _tc_body,
        grid=(NB_TC,),
        in_specs=[
            pl.BlockSpec((1, 1, BT), lambda i: (B0_TC + i, 0, 0)),
            pl.BlockSpec((BT, D), lambda i: (B0_TC + i, 0)),
            pl.BlockSpec((BT, D), lambda i: (B0_TC + i, 0)),
        ],
        out_specs=[
            pl.BlockSpec((NUM_SEG, D), lambda i: (0, 0)),
            pl.BlockSpec((NUM_SEG, D), lambda i: (0, 0)),
        ],
        out_shape=[
            jax.ShapeDtypeStruct((NUM_SEG, D), jnp.float32),
            jax.ShapeDtypeStruct((NUM_SEG, D), jnp.float32),
        ],
        scratch_shapes=[
            pltpu.VMEM((NUM_SEG, D), jnp.float32),
            pltpu.VMEM((NUM_SEG, D), jnp.float32),
        ],
    )(idx3, inp, tgt)


def _epi_body(psd_ref, pst_ref, td_ref, tt_ref, o_ref):
    sd = jnp.sum(psd_ref[...], axis=0).reshape(NUM_SEG, D) + td_ref[...]
    st = jnp.sum(pst_ref[...], axis=0).reshape(NUM_SEG, D) + tt_ref[...]
    dn = jnp.sqrt(sd)
    tn = jnp.maximum(jnp.sqrt(st), 1.0)
    o_ref[0, 0] = jnp.mean(dn / tn)


def _epilogue(psd, pst, td, tt):
    return pl.pallas_call(
        _epi_body,
        out_specs=pl.BlockSpec(memory_space=pltpu.SMEM),
        out_shape=jax.ShapeDtypeStruct((1, 1), jnp.float32),
    )(psd, pst, td, tt)


def kernel(input, target, batch_idx):
    idx32 = batch_idx.astype(jnp.int32)
    psd, pst = _seg_sumsq_sc(input, target, idx32)
    td, tt = _seg_sumsq_tc(input, target, idx32.reshape(TOTAL_TOK // BT, 1, BT))
    return _epilogue(psd, pst, td, tt)[0, 0]


# speedup vs baseline: 2.3068x; 1.2510x over previous
import functools

import jax
import jax.numpy as jnp
from jax.experimental import pallas as pl
from jax.experimental.pallas import tpu as pltpu

NUM_SEG = 16
TOTAL_TOK = 32768
D = 1024
BT = 2048
NB = TOTAL_TOK // BT


def _body(idx_ref, x_ref, t_ref, o_ref, acc_d, acc_t):
    i = pl.program_id(0)

    idx = idx_ref[0, 0, :]
    onehot = (jax.lax.broadcasted_iota(jnp.int32, (NUM_SEG, BT), 0)
              == idx[None, :]).astype(jnp.float32)

    x = x_ref[...]
    t = t_ref[...]
    d = x - t
    ds = d * d
    ts = t * t
    pd = jnp.dot(onehot, ds, preferred_element_type=jnp.float32)
    pt = jnp.dot(onehot, ts, preferred_element_type=jnp.float32)

    @pl.when(i == 0)
    def _init():
        acc_d[...] = pd
        acc_t[...] = pt

    @pl.when(i > 0)
    def _accum():
        acc_d[...] += pd
        acc_t[...] += pt

    @pl.when(i == NB - 1)
    def _fin():
        dn = jnp.sqrt(acc_d[...])
        tn = jnp.maximum(jnp.sqrt(acc_t[...]), 1.0)
        o_ref[0, 0] = jnp.mean(dn / tn)


@jax.jit
def _run(inp, tgt, idx3):
    out = pl.pallas_call(
        _body,
        grid=(NB,),
        in_specs=[
            pl.BlockSpec((1, 1, BT), lambda i: (i, 0, 0)),
            pl.BlockSpec((BT, D), lambda i: (i, 0)),
            pl.BlockSpec((BT, D), lambda i: (i, 0)),
        ],
        out_specs=pl.BlockSpec((1, 1), lambda i: (0, 0), memory_space=pltpu.SMEM),
        out_shape=jax.ShapeDtypeStruct((1, 1), jnp.float32),
        scratch_shapes=[
            pltpu.VMEM((NUM_SEG, D), jnp.float32),
            pltpu.VMEM((NUM_SEG, D), jnp.float32),
        ],
    )(idx3, inp, tgt)
    return out[0, 0]


def kernel(input, target, batch_idx):
    idx3 = batch_idx.astype(jnp.int32).reshape(NB, 1, BT)
    return _run(input, target, idx3)
